# baseline (speedup 1.0000x reference)
import functools

import jax
import jax.numpy as jnp
from jax.experimental import pallas as pl
from jax.experimental.pallas import tpu as pltpu

_COORDS_RANGE = 10.0


def _silu(v):
    return v * jax.nn.sigmoid(v)


def _edge_body(gA, gB, geom, wr, b1, W2T, b2, cW1T, cb1, cW2r,
               msgh_out, small_out):
    r = geom[:, 3:4]
    pre = gA[:] + gB[:] + r * wr[:] + b1[:]
    m = _silu(pre)
    mh = _silu(jnp.dot(m, W2T[:], preferred_element_type=jnp.float32) + b2[:])
    c = _silu(jnp.dot(mh, cW1T[:], preferred_element_type=jnp.float32) + cb1[:])
    t = jnp.tanh(jnp.sum(c * cW2r[:], axis=1, keepdims=True))
    msgh_out[:] = mh
    u = geom[:, 0:3]
    mx = t * u * _COORDS_RANGE
    ones = jnp.ones_like(t)
    small_out[:] = jnp.concatenate(
        [mx, ones, jnp.zeros((t.shape[0], 4), t.dtype)], axis=1)


def _edge_mlp(gA, gB, geom, p):
    E = gA.shape[0]
    H = gA.shape[1]
    blk = 2560 if E % 2560 == 0 else E
    nblk = E // blk
    wr = p['eW1'][:, -1].reshape(1, H)
    b1 = p['eb1'].reshape(1, H)
    W2T = p['eW2'].T
    b2 = p['eb2'].reshape(1, H)
    cW1T = p['cW1'].T
    cb1 = p['cb1'].reshape(1, H)
    cW2r = p['cW2'].reshape(1, H)
    full = lambda shape: pl.BlockSpec(shape, lambda i: (0, 0))
    msgh, small = pl.pallas_call(
        _edge_body,
        grid=(nblk,),
        in_specs=[
            pl.BlockSpec((blk, H), lambda i: (i, 0)),
            pl.BlockSpec((blk, H), lambda i: (i, 0)),
            pl.BlockSpec((blk, 8), lambda i: (i, 0)),
            full((1, H)), full((1, H)), full((H, H)), full((1, H)),
            full((H, H)), full((1, H)), full((1, H)),
        ],
        out_specs=[
            pl.BlockSpec((blk, H), lambda i: (i, 0)),
            pl.BlockSpec((blk, 8), lambda i: (i, 0)),
        ],
        out_shape=[
            jax.ShapeDtypeStruct((E, H), jnp.float32),
            jax.ShapeDtypeStruct((E, 8), jnp.float32),
        ],
    )(gA, gB, geom, wr, b1, W2T, b2, cW1T, cb1, cW2r)
    return msgh, small


def _proj_body(h, WaT, WbT, eA_out, eB_out):
    eA_out[:] = jnp.dot(h[:], WaT[:], preferred_element_type=jnp.float32)
    eB_out[:] = jnp.dot(h[:], WbT[:], preferred_element_type=jnp.float32)


def _edge_proj(h, eW1):
    N, F = h.shape
    H = eW1.shape[0]
    WaT = eW1[:, :F].T
    WbT = eW1[:, F:2 * F].T
    Fp = max(8, -(-F // 8) * 8)
    if Fp != F:
        h = jnp.pad(h, ((0, 0), (0, Fp - F)))
        WaT = jnp.pad(WaT, ((0, Fp - F), (0, 0)))
        WbT = jnp.pad(WbT, ((0, Fp - F), (0, 0)))
    blk = 2000 if N % 2000 == 0 else N
    nblk = N // blk
    eA, eB = pl.pallas_call(
        _proj_body,
        grid=(nblk,),
        in_specs=[
            pl.BlockSpec((blk, Fp), lambda i: (i, 0)),
            pl.BlockSpec((Fp, H), lambda i: (0, 0)),
            pl.BlockSpec((Fp, H), lambda i: (0, 0)),
        ],
        out_specs=[
            pl.BlockSpec((blk, H), lambda i: (i, 0)),
            pl.BlockSpec((blk, H), lambda i: (i, 0)),
        ],
        out_shape=[
            jax.ShapeDtypeStruct((N, H), jnp.float32),
            jax.ShapeDtypeStruct((N, H), jnp.float32),
        ],
    )(h, WaT, WbT)
    return eA, eB


def _node_body(h, hn, W1aT, W1bT, b1, W2T, b2, hout):
    z = (jnp.dot(h[:], W1aT[:], preferred_element_type=jnp.float32)
         + jnp.dot(hn[:], W1bT[:], preferred_element_type=jnp.float32) + b1[:])
    z = _silu(z)
    hout[:] = jnp.dot(z, W2T[:], preferred_element_type=jnp.float32) + b2[:]


def _node_update(h, h_neigh, p):
    N, F = h.shape
    H = h_neigh.shape[1]
    OUTF = p['nW2'].shape[0]
    W1aT = p['nW1'][:, :F].T
    W1bT = p['nW1'][:, F:].T
    b1 = p['nb1'].reshape(1, -1)
    W2T = p['nW2'].T
    b2 = p['nb2'].reshape(1, -1)
    Fp = max(8, -(-F // 8) * 8)
    if Fp != F:
        h = jnp.pad(h, ((0, 0), (0, Fp - F)))
        W1aT = jnp.pad(W1aT, ((0, Fp - F), (0, 0)))
    blk = 2000 if N % 2000 == 0 else N
    nblk = N // blk
    hout = pl.pallas_call(
        _node_body,
        grid=(nblk,),
        in_specs=[
            pl.BlockSpec((blk, Fp), lambda i: (i, 0)),
            pl.BlockSpec((blk, H), lambda i: (i, 0)),
            pl.BlockSpec((Fp, H), lambda i: (0, 0)),
            pl.BlockSpec((H, H), lambda i: (0, 0)),
            pl.BlockSpec((1, H), lambda i: (0, 0)),
            pl.BlockSpec((H, OUTF), lambda i: (0, 0)),
            pl.BlockSpec((1, OUTF), lambda i: (0, 0)),
        ],
        out_specs=pl.BlockSpec((blk, OUTF), lambda i: (i, 0)),
        out_shape=jax.ShapeDtypeStruct((N, OUTF), jnp.float32),
    )(h, h_neigh, W1aT, W1bT, b1, W2T, b2)
    return hout


def kernel(node_feat, coord_feat, edge_index, params):
    src = edge_index[0].astype(jnp.int32)
    dst = edge_index[1].astype(jnp.int32)
    N = node_feat.shape[0]
    h, x = node_feat, coord_feat
    for p in params:
        xs = jnp.take(x, src, axis=0)
        xd = jnp.take(x, dst, axis=0)
        xdiff = xs - xd
        radial = jnp.sum(xdiff * xdiff, axis=1, keepdims=True)
        u = xdiff / (jnp.sqrt(radial) + 1e-30)
        geom = jnp.concatenate(
            [u, radial, jnp.zeros((u.shape[0], 4), u.dtype)], axis=1)
        eA, eB = _edge_proj(h, p['eW1'])
        gA = jnp.take(eA, src, axis=0)
        gB = jnp.take(eB, dst, axis=0)
        msgh, small = _edge_mlp(gA, gB, geom, p)
        h_neigh = jax.ops.segment_sum(msgh, dst, num_segments=N)
        sm = jax.ops.segment_sum(small, dst, num_segments=N)
        deg = jnp.maximum(sm[:, 3:4], 1.0)
        x = x + sm[:, 0:3] / deg
        h = _node_update(h, h_neigh, p)
    return (h, x)


# speedup vs baseline: 3.6799x; 3.6799x over previous
import functools

import jax
import jax.numpy as jnp
from jax import lax
from jax.experimental import pallas as pl
from jax.experimental.pallas import tpu as pltpu
from jax.experimental.pallas import tpu_sc as plsc

_COORDS_RANGE = 10.0
_NC = 2
_NS = 16
_NW = _NC * _NS
_C = 128


def _silu(v):
    return v * jax.nn.sigmoid(v)


def _mesh():
    return plsc.VectorSubcoreMesh(core_axis_name="c", subcore_axis_name="s",
                                  num_cores=_NC, num_subcores=_NS)


_SC_PARAMS = pltpu.CompilerParams(use_tc_tiling_on_sc=False)


def _gather1_body(K, tab, idxs3, idxd3, gA_out, gB_out,
                  ivs, ivd, bufA, bufB, sem0, sem1):
    cid = lax.axis_index("c")
    sid = lax.axis_index("s")
    wid = cid * _NS + sid
    base = wid * (K * _C)
    pltpu.sync_copy(idxs3.at[wid], ivs)
    pltpu.sync_copy(idxd3.at[wid], ivd)
    sems = (sem0, sem1)

    def issue(j, b):
        pltpu.async_copy(tab.at[ivs.at[j]], bufA.at[b], sems[b])
        pltpu.async_copy(tab.at[ivd.at[j]], bufB.at[b], sems[b])

    def drain(b):
        pltpu.make_async_copy(tab.at[pl.ds(0, _C)], bufA.at[b], sems[b]).wait()
        pltpu.make_async_copy(tab.at[pl.ds(0, _C)], bufB.at[b], sems[b]).wait()

    issue(0, 0)

    def step(i2, _):
        for b in (0, 1):
            j = i2 * 2 + b

            @pl.when(j + 1 < K)
            def _():
                issue(j + 1, 1 - b)

            drain(b)
            pltpu.sync_copy(bufA.at[b], gA_out.at[pl.ds(base + j * _C, _C)])
            pltpu.sync_copy(bufB.at[b], gB_out.at[pl.ds(base + j * _C, _C)])
        return _

    lax.fori_loop(0, K // 2, step, None)


def _sc_gather1(tab, idxs3, idxd3, K, Ep):
    D = tab.shape[1]
    kfn = pl.kernel(
        functools.partial(_gather1_body, K),
        out_type=[jax.ShapeDtypeStruct((Ep, D), jnp.float32),
                  jax.ShapeDtypeStruct((Ep, D), jnp.float32)],
        mesh=_mesh(),
        compiler_params=_SC_PARAMS,
        scratch_types=[
            pltpu.VMEM((K, _C), jnp.int32),
            pltpu.VMEM((K, _C), jnp.int32),
            pltpu.VMEM((2, _C, D), jnp.float32),
            pltpu.VMEM((2, _C, D), jnp.float32),
            pltpu.SemaphoreType.DMA, pltpu.SemaphoreType.DMA,
        ],
    )
    return kfn(tab, idxs3, idxd3)


def _gather2_body(K, eA, eB, XP, idxs3, idxd3, gsum_out, gx_out,
                  ivs, ivd, bufA, bufB, bufXS, bufXD, sem0, sem1):
    cid = lax.axis_index("c")
    sid = lax.axis_index("s")
    wid = cid * _NS + sid
    base = wid * (K * _C)
    pltpu.sync_copy(idxs3.at[wid], ivs)
    pltpu.sync_copy(idxd3.at[wid], ivd)
    sems = (sem0, sem1)

    def issue(j, b):
        pltpu.async_copy(eA.at[ivs.at[j]], bufA.at[b], sems[b])
        pltpu.async_copy(eB.at[ivd.at[j]], bufB.at[b], sems[b])
        pltpu.async_copy(XP.at[ivs.at[j]], bufXS.at[b], sems[b])
        pltpu.async_copy(XP.at[ivd.at[j]], bufXD.at[b], sems[b])

    def drain(b):
        pltpu.make_async_copy(eA.at[pl.ds(0, _C)], bufA.at[b], sems[b]).wait()
        pltpu.make_async_copy(eB.at[pl.ds(0, _C)], bufB.at[b], sems[b]).wait()
        pltpu.make_async_copy(XP.at[pl.ds(0, _C)], bufXS.at[b], sems[b]).wait()
        pltpu.make_async_copy(XP.at[pl.ds(0, _C)], bufXD.at[b], sems[b]).wait()

    issue(0, 0)

    def step(i2, _):
        for b in (0, 1):
            j = i2 * 2 + b

            @pl.when(j + 1 < K)
            def _():
                issue(j + 1, 1 - b)

            drain(b)

            def row(r, carry):
                for k in range(8):
                    plsc.addupdate(bufA.at[b, r, pl.ds(16 * k, 16)],
                                   bufB[b, r, pl.ds(16 * k, 16)])
                bufXS[b, r, :] = bufXS[b, r, :] - bufXD[b, r, :]
                return carry

            lax.fori_loop(0, _C, row, None)
            pltpu.sync_copy(bufA.at[b], gsum_out.at[pl.ds(base + j * _C, _C)])
            pltpu.sync_copy(bufXS.at[b], gx_out.at[pl.ds(base + j * _C, _C)])
        return _

    lax.fori_loop(0, K // 2, step, None)


def _sc_gather2(eA, eB, XP, idxs3, idxd3, K, Ep):
    H = eA.shape[1]
    kfn = pl.kernel(
        functools.partial(_gather2_body, K),
        out_type=[jax.ShapeDtypeStruct((Ep, H), jnp.float32),
                  jax.ShapeDtypeStruct((Ep, 16), jnp.float32)],
        mesh=_mesh(),
        compiler_params=_SC_PARAMS,
        scratch_types=[
            pltpu.VMEM((K, _C), jnp.int32),
            pltpu.VMEM((K, _C), jnp.int32),
            pltpu.VMEM((2, _C, H), jnp.float32),
            pltpu.VMEM((2, _C, H), jnp.float32),
            pltpu.VMEM((2, _C, 16), jnp.float32),
            pltpu.VMEM((2, _C, 16), jnp.float32),
            pltpu.SemaphoreType.DMA, pltpu.SemaphoreType.DMA,
        ],
    )
    return kfn(eA, eB, XP, idxs3, idxd3)


def _scatter_body(K, Np, msgh, small, idxd3, zh, zs, hacc_out, sacc_out,
                  ivd, bufh, bufs, acc_h, acc_s, sem0, sem1):
    cid = lax.axis_index("c")
    sid = lax.axis_index("s")
    wid = cid * _NS + sid
    base = wid * (K * _C)
    rows = Np // _NS
    zbase = sid * rows
    nfull = rows // _C
    for t in range(nfull):
        pltpu.sync_copy(zh.at[pl.ds(0, _C)], acc_h.at[pl.ds(zbase + t * _C, _C)])
        pltpu.sync_copy(zs.at[pl.ds(0, _C)], acc_s.at[pl.ds(zbase + t * _C, _C)])
    rem = rows - nfull * _C
    if rem:
        pltpu.sync_copy(zh.at[pl.ds(0, rem)],
                        acc_h.at[pl.ds(zbase + nfull * _C, rem)])
        pltpu.sync_copy(zs.at[pl.ds(0, rem)],
                        acc_s.at[pl.ds(zbase + nfull * _C, rem)])
    plsc.subcore_barrier()

    pltpu.sync_copy(idxd3.at[wid], ivd)
    sems = (sem0, sem1)

    def issue(j, b):
        pltpu.async_copy(msgh.at[pl.ds(base + j * _C, _C)], bufh.at[b], sems[b])
        pltpu.async_copy(small.at[pl.ds(base + j * _C, _C)], bufs.at[b], sems[b])

    def drain(b):
        pltpu.make_async_copy(msgh.at[pl.ds(0, _C)], bufh.at[b], sems[b]).wait()
        pltpu.make_async_copy(small.at[pl.ds(0, _C)], bufs.at[b], sems[b]).wait()

    issue(0, 0)

    def step(i2, _):
        for b in (0, 1):
            j = i2 * 2 + b

            @pl.when(j + 1 < K)
            def _():
                issue(j + 1, 1 - b)

            drain(b)
            pltpu.sync_copy(bufh.at[b], acc_h.at[ivd.at[j]], add=True)
            pltpu.sync_copy(bufs.at[b], acc_s.at[ivd.at[j]], add=True)
        return _

    lax.fori_loop(0, K // 2, step, None)
    plsc.subcore_barrier()
    pltpu.sync_copy(acc_h.at[pl.ds(zbase, rows)],
                    hacc_out.at[cid, pl.ds(zbase, rows)])
    pltpu.sync_copy(acc_s.at[pl.ds(zbase, rows)],
                    sacc_out.at[cid, pl.ds(zbase, rows)])


def _sc_scatter(msgh, small, idxd3, zh, zs, K, Np):
    H = msgh.shape[1]
    kfn = pl.kernel(
        functools.partial(_scatter_body, K, Np),
        out_type=[jax.ShapeDtypeStruct((_NC, Np, H), jnp.float32),
                  jax.ShapeDtypeStruct((_NC, Np, 8), jnp.float32)],
        mesh=_mesh(),
        compiler_params=_SC_PARAMS,
        scratch_types=[
            pltpu.VMEM((K, _C), jnp.int32),
            pltpu.VMEM((2, _C, H), jnp.float32),
            pltpu.VMEM((2, _C, 8), jnp.float32),
            pltpu.VMEM_SHARED((Np, H), jnp.float32),
            pltpu.VMEM_SHARED((Np, 8), jnp.float32),
            pltpu.SemaphoreType.DMA, pltpu.SemaphoreType.DMA,
        ],
    )
    return kfn(msgh, small, idxd3, zh, zs)


def _mlp_tail(pre, u, W2T, b2, cW1T, cb1, cW2r, msgh_out, small_out):
    m = _silu(pre)
    mh = _silu(jnp.dot(m, W2T[:], preferred_element_type=jnp.float32) + b2[:])
    c = _silu(jnp.dot(mh, cW1T[:], preferred_element_type=jnp.float32) + cb1[:])
    t = jnp.tanh(jnp.sum(c * cW2r[:], axis=1, keepdims=True))
    msgh_out[:] = mh
    mx = t * u * _COORDS_RANGE
    ones = jnp.ones_like(t)
    small_out[:] = jnp.concatenate(
        [mx, ones, jnp.zeros((t.shape[0], 4), t.dtype)], axis=1)


def _edge_body1(F, gA, gB, WaT, WbT, wr, b1, W2T, b2, cW1T, cb1, cW2r,
                msgh_out, small_out):
    xdiff = gA[:, F:F + 3] - gB[:, F:F + 3]
    radial = jnp.sum(xdiff * xdiff, axis=1, keepdims=True)
    u = xdiff / (jnp.sqrt(radial) + 1e-30)
    pre = (jnp.dot(gA[:], WaT[:], preferred_element_type=jnp.float32)
           + jnp.dot(gB[:], WbT[:], preferred_element_type=jnp.float32)
           + radial * wr[:] + b1[:])
    _mlp_tail(pre, u, W2T, b2, cW1T, cb1, cW2r, msgh_out, small_out)


def _edge_mlp1(gA, gB, p, F):
    Ep, Dp = gA.shape
    H = p['eW2'].shape[0]
    WaT = jnp.pad(p['eW1'][:, :F].T, ((0, Dp - F), (0, 0)))
    WbT = jnp.pad(p['eW1'][:, F:2 * F].T, ((0, Dp - F), (0, 0)))
    wr = p['eW1'][:, 2 * F].reshape(1, H)
    b1 = p['eb1'].reshape(1, H)
    blk = 2560
    full = lambda shape: pl.BlockSpec(shape, lambda i: (0, 0))
    return pl.pallas_call(
        functools.partial(_edge_body1, F),
        grid=(Ep // blk,),
        in_specs=[
            pl.BlockSpec((blk, Dp), lambda i: (i, 0)),
            pl.BlockSpec((blk, Dp), lambda i: (i, 0)),
            full((Dp, H)), full((Dp, H)), full((1, H)), full((1, H)),
            full((H, H)), full((1, H)), full((H, H)), full((1, H)),
            full((1, H)),
        ],
        out_specs=[
            pl.BlockSpec((blk, H), lambda i: (i, 0)),
            pl.BlockSpec((blk, 8), lambda i: (i, 0)),
        ],
        out_shape=[
            jax.ShapeDtypeStruct((Ep, H), jnp.float32),
            jax.ShapeDtypeStruct((Ep, 8), jnp.float32),
        ],
    )(gA, gB, WaT, WbT, wr, b1, p['eW2'].T, p['eb2'].reshape(1, H),
      p['cW1'].T, p['cb1'].reshape(1, H), p['cW2'].reshape(1, H))


def _edge_body2(gsum, gx, wr, b1, W2T, b2, cW1T, cb1, cW2r,
                msgh_out, small_out):
    xdiff = gx[:, 0:3]
    radial = jnp.sum(xdiff * xdiff, axis=1, keepdims=True)
    u = xdiff / (jnp.sqrt(radial) + 1e-30)
    pre = gsum[:] + radial * wr[:] + b1[:]
    _mlp_tail(pre, u, W2T, b2, cW1T, cb1, cW2r, msgh_out, small_out)


def _edge_mlp2(gsum, gx, p):
    Ep, H = gsum.shape
    wr = p['eW1'][:, -1].reshape(1, H)
    b1 = p['eb1'].reshape(1, H)
    blk = 2560
    full = lambda shape: pl.BlockSpec(shape, lambda i: (0, 0))
    return pl.pallas_call(
        _edge_body2,
        grid=(Ep // blk,),
        in_specs=[
            pl.BlockSpec((blk, H), lambda i: (i, 0)),
            pl.BlockSpec((blk, 16), lambda i: (i, 0)),
            full((1, H)), full((1, H)), full((H, H)), full((1, H)),
            full((H, H)), full((1, H)), full((1, H)),
        ],
        out_specs=[
            pl.BlockSpec((blk, H), lambda i: (i, 0)),
            pl.BlockSpec((blk, 8), lambda i: (i, 0)),
        ],
        out_shape=[
            jax.ShapeDtypeStruct((Ep, H), jnp.float32),
            jax.ShapeDtypeStruct((Ep, 8), jnp.float32),
        ],
    )(gsum, gx, wr, b1, p['eW2'].T, p['eb2'].reshape(1, H),
      p['cW1'].T, p['cb1'].reshape(1, H), p['cW2'].reshape(1, H))


def _proj_body(h, WaT, WbT, eA_out, eB_out):
    eA_out[:] = jnp.dot(h[:], WaT[:], preferred_element_type=jnp.float32)
    eB_out[:] = jnp.dot(h[:], WbT[:], preferred_element_type=jnp.float32)


def _edge_proj(h, eW1):
    N, F = h.shape
    H = eW1.shape[0]
    WaT = eW1[:, :F].T
    WbT = eW1[:, F:2 * F].T
    blk = 2000 if N % 2000 == 0 else N
    return pl.pallas_call(
        _proj_body,
        grid=(N // blk,),
        in_specs=[
            pl.BlockSpec((blk, F), lambda i: (i, 0)),
            pl.BlockSpec((F, H), lambda i: (0, 0)),
            pl.BlockSpec((F, H), lambda i: (0, 0)),
        ],
        out_specs=[
            pl.BlockSpec((blk, H), lambda i: (i, 0)),
            pl.BlockSpec((blk, H), lambda i: (i, 0)),
        ],
        out_shape=[
            jax.ShapeDtypeStruct((N, H), jnp.float32),
            jax.ShapeDtypeStruct((N, H), jnp.float32),
        ],
    )(h, WaT, WbT)


def _node_body(h, hacc, W1aT, W1bT, b1, W2T, b2, hout):
    hn = hacc[0] + hacc[1]
    z = (jnp.dot(h[:], W1aT[:], preferred_element_type=jnp.float32)
         + jnp.dot(hn, W1bT[:], preferred_element_type=jnp.float32) + b1[:])
    z = _silu(z)
    hout[:] = jnp.dot(z, W2T[:], preferred_element_type=jnp.float32) + b2[:]


def _node_update(h, hacc, p):
    N, F = h.shape
    H = hacc.shape[2]
    OUTF = p['nW2'].shape[0]
    W1aT = p['nW1'][:, :F].T
    W1bT = p['nW1'][:, F:].T
    b1 = p['nb1'].reshape(1, -1)
    W2T = p['nW2'].T
    b2 = p['nb2'].reshape(1, -1)
    Fp = max(8, -(-F // 8) * 8)
    if Fp != F:
        h = jnp.pad(h, ((0, 0), (0, Fp - F)))
        W1aT = jnp.pad(W1aT, ((0, Fp - F), (0, 0)))
    blk = 2000 if N % 2000 == 0 else N
    return pl.pallas_call(
        _node_body,
        grid=(N // blk,),
        in_specs=[
            pl.BlockSpec((blk, Fp), lambda i: (i, 0)),
            pl.BlockSpec((2, blk, H), lambda i: (0, i, 0)),
            pl.BlockSpec((Fp, H), lambda i: (0, 0)),
            pl.BlockSpec((H, H), lambda i: (0, 0)),
            pl.BlockSpec((1, H), lambda i: (0, 0)),
            pl.BlockSpec((H, OUTF), lambda i: (0, 0)),
            pl.BlockSpec((1, OUTF), lambda i: (0, 0)),
        ],
        out_specs=pl.BlockSpec((blk, OUTF), lambda i: (i, 0)),
        out_shape=jax.ShapeDtypeStruct((N, OUTF), jnp.float32),
    )(h, hacc, W1aT, W1bT, b1, W2T, b2)


def kernel(node_feat, coord_feat, edge_index, params):
    src = edge_index[0].astype(jnp.int32)
    dst = edge_index[1].astype(jnp.int32)
    N = node_feat.shape[0]
    E = src.shape[0]
    F = node_feat.shape[1]

    K = -(-E // (_NW * _C))
    K = -(-K // 10) * 10
    Ep = _NW * _C * K
    Np = -(-(N + 1) // _C) * _C

    pad = Ep - E
    srcg3 = jnp.pad(src, (0, pad)).reshape(_NW, K, _C)
    dstg3 = jnp.pad(dst, (0, pad)).reshape(_NW, K, _C)
    dsts3 = jnp.pad(dst, (0, pad), constant_values=N).reshape(_NW, K, _C)
    zh = jnp.zeros((_C, 128), jnp.float32)
    zs = jnp.zeros((_C, 8), jnp.float32)

    h, x = node_feat, coord_feat
    for li, p in enumerate(params):
        if li == 0:
            pack = jnp.concatenate(
                [h, jnp.zeros((N, 16 - F - 3), jnp.float32), x], axis=1)
            gA, gB = _sc_gather1(pack, srcg3, dstg3, K, Ep)
            msgh, small = _edge_mlp1(gA, gB, p, F)
        else:
            eA, eB = _edge_proj(h, p['eW1'])
            XP = jnp.pad(x, ((0, 0), (0, 13)))
            gsum, gx = _sc_gather2(eA, eB, XP, srcg3, dstg3, K, Ep)
            msgh, small = _edge_mlp2(gsum, gx, p)
        hacc, sacc = _sc_scatter(msgh, small, dsts3, zh, zs, K, Np)
        sm = sacc[0, :N] + sacc[1, :N]
        deg = jnp.maximum(sm[:, 3:4], 1.0)
        x = x + sm[:, 0:3] / deg
        h = _node_update(h, hacc, p)
    return (h, x)
